# baseline (speedup 1.0000x reference)
import functools

import jax
import jax.numpy as jnp
from jax import lax
from jax.experimental import pallas as pl
from jax.experimental.pallas import tpu as pltpu
from jax.experimental.pallas import tpu_sc as plsc

_NC = 2
_NS = 16
_NW = _NC * _NS
_NBUF = 2


@functools.lru_cache(maxsize=None)
def _make_gather(b, t, vocab, tp, vp, ti, vi):
    per_w = b // _NW
    assert per_w * _NW == b and per_w % _NBUF == 0
    mesh = plsc.VectorSubcoreMesh(core_axis_name="c", subcore_axis_name="s")

    @functools.partial(
        pl.kernel,
        mesh=mesh,
        out_type=(
            jax.ShapeDtypeStruct((b, t, vocab), jnp.float32),
            jax.ShapeDtypeStruct((b, tp - ti, vp), jnp.float32),
            jax.ShapeDtypeStruct((b, ti, vp - vi), jnp.float32),
        ),
        scratch_types=[
            pltpu.VMEM((per_w, 128), jnp.int32),
            [pltpu.VMEM((tp, vp), jnp.float32) for _ in range(_NBUF)],
            [pltpu.SemaphoreType.DMA for _ in range(_NBUF)],
            [pltpu.SemaphoreType.DMA for _ in range(_NBUF)],
        ],
    )
    def gather(idx_hbm, table_hbm, out_hbm, srow_hbm, scol_hbm,
               idx_v, rows, gsem, wsem):
        wid = lax.axis_index("s") * _NC + lax.axis_index("c")
        b0 = pl.multiple_of(wid * per_w, 8)
        pltpu.sync_copy(idx_hbm.at[pl.ds(b0, per_w)], idx_v)

        def issue_gather(c, buf):
            pltpu.async_copy(table_hbm.at[idx_v.at[c, pl.ds(0, tp)]],
                             rows[buf], gsem[buf])

        def wait_gather(c, buf):
            pltpu.make_async_copy(table_hbm.at[idx_v.at[c, pl.ds(0, tp)]],
                                  rows[buf], gsem[buf]).wait()

        def write_descs(c, buf):
            return (
                (rows[buf].at[pl.ds(0, ti), pl.ds(0, vi)],
                 out_hbm.at[b0 + c, pl.ds(0, ti), pl.ds(0, vi)]),
                (rows[buf].at[pl.ds(ti, tp - ti)], srow_hbm.at[b0 + c]),
                (rows[buf].at[pl.ds(0, ti), pl.ds(vi, vp - vi)],
                 scol_hbm.at[b0 + c]),
            )

        def issue_writes(c, buf):
            for src, dst in write_descs(c, buf):
                pltpu.async_copy(src, dst, wsem[buf])

        def wait_writes(c, buf):
            for src, dst in write_descs(c, buf):
                pltpu.make_async_copy(src, dst, wsem[buf]).wait()

        @pl.loop(0, per_w, step=_NBUF)
        def _body(c0):
            for bb in range(_NBUF):
                c = c0 + bb

                @pl.when(c >= _NBUF)
                def _():
                    wait_writes(c - _NBUF, bb)

                issue_gather(c, bb)

                @pl.when(c >= 1)
                def _():
                    wait_gather(c - 1, (bb - 1) % _NBUF)
                    issue_writes(c - 1, (bb - 1) % _NBUF)

        last = per_w - 1
        wait_gather(last, last % _NBUF)
        issue_writes(last, last % _NBUF)
        for c in range(per_w - _NBUF, per_w):
            wait_writes(c, c % _NBUF)

    return gather


def _copy_body(_, strip_ref, o_ref):
    o_ref[...] = strip_ref[...]


def _strip_pass(out_prev, strip, block, out_block_idx, grid_b):
    b, t, vocab = out_prev.shape
    return pl.pallas_call(
        _copy_body,
        grid=(b // grid_b,),
        in_specs=[
            pl.BlockSpec(memory_space=pl.ANY),
            pl.BlockSpec((grid_b,) + block, lambda g: (g, 0, 0)),
        ],
        out_specs=pl.BlockSpec((grid_b,) + block,
                               lambda g, _j=out_block_idx: (g, _j[0], _j[1])),
        out_shape=jax.ShapeDtypeStruct((b, t, vocab), jnp.float32),
        input_output_aliases={0: 0},
    )(out_prev, strip)


def kernel(idx, table):
    b, t = idx.shape
    vocab = table.shape[1]
    tp = (t + 7) // 8 * 8
    vp = (vocab + 127) // 128 * 128
    ti = t // 8 * 8
    vi = vocab // 128 * 128
    idx_p = jnp.pad(idx.astype(jnp.int32), ((0, 0), (0, 128 - t)))
    table_p = jnp.pad(table, ((0, 0), (0, vp - vocab)))
    out, srow, scol = _make_gather(b, t, vocab, tp, vp, ti, vi)(idx_p, table_p)
    out = _strip_pass(out, srow, (tp - ti, vp), (ti // (tp - ti), 0), 64)
    out = _strip_pass(out, scol, (ti, vp - vi), (0, vi // (vp - vi)), 64)
    return out
